# baseline (device time: 16503 ns/iter reference)
import jax
import jax.numpy as jnp
from jax import lax
from jax.experimental import pallas as pl
from jax.experimental.pallas import tpu as pltpu

NB = 8


def kernel(x):
    m, n = x.shape
    out_cols = n // 2
    half = m // 2
    br = half // NB

    def body(x_hbm, out_ref, send_src, own_buf, send_stage, local_sems,
             x_send_sems, x_recv_sems, y_send_sems, y_recv_sems):
        my_x = lax.axis_index("x")
        my_y = lax.axis_index("y")
        other_x = 1 - my_x
        other_y = 1 - my_y

        barrier_sem = pltpu.get_barrier_semaphore()
        for dev in [(other_x, my_y), (my_x, other_y)]:
            pl.semaphore_signal(
                barrier_sem, inc=1, device_id=dev,
                device_id_type=pl.DeviceIdType.MESH,
            )

        cp_send = pltpu.make_async_copy(
            x_hbm.at[pl.ds(my_y * half, half),
                     pl.ds(other_x * out_cols, out_cols)],
            send_src,
            local_sems.at[0],
        )
        cp_send.start()
        cp_own = pltpu.make_async_copy(
            x_hbm.at[:, pl.ds(my_x * out_cols, out_cols)],
            own_buf,
            local_sems.at[1],
        )
        cp_own.start()

        cp_send.wait()
        pl.semaphore_wait(barrier_sem, 2)

        dst_base = my_x * m + my_y * half
        recv_base = other_x * m + my_y * half

        x_rdmas = []
        for b in range(NB):
            rows = pl.ds(b * br, br)
            send_stage[rows, :] = send_src[rows, :].astype(jnp.bfloat16)
            rdma = pltpu.make_async_remote_copy(
                src_ref=send_stage.at[rows],
                dst_ref=out_ref.at[pl.ds(dst_base + b * br, br)],
                send_sem=x_send_sems.at[b],
                recv_sem=x_recv_sems.at[b],
                device_id=(other_x, my_y),
                device_id_type=pl.DeviceIdType.MESH,
            )
            rdma.start()
            x_rdmas.append(rdma)

        cp_own.wait()
        out_ref[pl.ds(my_x * m, m), :] = own_buf[...].astype(jnp.bfloat16)

        y_rdmas = []
        for b in range(NB):
            x_rdmas[b].wait_recv()
            rdma = pltpu.make_async_remote_copy(
                src_ref=out_ref.at[pl.ds(recv_base + b * br, br)],
                dst_ref=out_ref.at[pl.ds(recv_base + b * br, br)],
                send_sem=y_send_sems.at[b],
                recv_sem=y_recv_sems.at[b],
                device_id=(my_x, other_y),
                device_id_type=pl.DeviceIdType.MESH,
            )
            rdma.start()
            y_rdmas.append(rdma)

        for b in range(NB):
            y_rdmas[b].wait_recv()
        for b in range(NB):
            x_rdmas[b].wait_send()
            y_rdmas[b].wait_send()

    out_shape = jax.ShapeDtypeStruct((2 * m, out_cols), jnp.bfloat16)
    return pl.pallas_call(
        body,
        out_shape=out_shape,
        in_specs=[pl.BlockSpec(memory_space=pl.ANY)],
        out_specs=pl.BlockSpec(memory_space=pltpu.VMEM),
        scratch_shapes=[
            pltpu.VMEM((half, out_cols), jnp.float32),
            pltpu.VMEM((m, out_cols), jnp.float32),
            pltpu.VMEM((half, out_cols), jnp.bfloat16),
            pltpu.SemaphoreType.DMA((2,)),
            pltpu.SemaphoreType.DMA((NB,)),
            pltpu.SemaphoreType.DMA((NB,)),
            pltpu.SemaphoreType.DMA((NB,)),
            pltpu.SemaphoreType.DMA((NB,)),
        ],
        compiler_params=pltpu.CompilerParams(collective_id=0),
    )(x)


# device time: 14877 ns/iter; 1.1093x vs baseline; 1.1093x over previous
import jax
import jax.numpy as jnp
from jax import lax
from jax.experimental import pallas as pl
from jax.experimental.pallas import tpu as pltpu

BLOCKS = (32,) * 14 + (16, 16)
NB = len(BLOCKS)
OFFS = tuple(sum(BLOCKS[:i]) for i in range(NB))


def kernel(x):
    x = pltpu.with_memory_space_constraint(x, pltpu.MemorySpace.HBM)
    m, n = x.shape
    out_cols = n // 2
    half = m // 2

    def body(x_hbm, out_hbm, send_src, own_src, own_bf16, send_stage,
             xbuf, lsems, lcp_sems, x_send_sems, x_recv_sems,
             y_send_sems, y_recv_sems):
        my_x = lax.axis_index("x")
        my_y = lax.axis_index("y")
        other_x = 1 - my_x
        other_y = 1 - my_y

        barrier_sem = pltpu.get_barrier_semaphore()
        for dev in [(other_x, my_y), (my_x, other_y)]:
            pl.semaphore_signal(
                barrier_sem, inc=1, device_id=dev,
                device_id_type=pl.DeviceIdType.MESH,
            )

        cp_send = pltpu.make_async_copy(
            x_hbm.at[pl.ds(my_y * half, half),
                     pl.ds(other_x * out_cols, out_cols)],
            send_src, lsems.at[0],
        )
        cp_send.start()
        cp_own = pltpu.make_async_copy(
            x_hbm.at[:, pl.ds(my_x * out_cols, out_cols)],
            own_src, lsems.at[1],
        )
        cp_own.start()

        cp_send.wait()
        send_stage[...] = send_src[...].astype(jnp.bfloat16)
        pl.semaphore_wait(barrier_sem, 2)

        recv_base = other_x * m + my_y * half

        x_rdmas = []
        for b in range(NB):
            br = BLOCKS[b]
            rdma = pltpu.make_async_remote_copy(
                src_ref=send_stage.at[pl.ds(OFFS[b], br)],
                dst_ref=xbuf.at[pl.ds(OFFS[b], br)],
                send_sem=x_send_sems.at[b],
                recv_sem=x_recv_sems.at[b],
                device_id=(other_x, my_y),
                device_id_type=pl.DeviceIdType.MESH,
            )
            rdma.start()
            x_rdmas.append(rdma)

        cp_own.wait()
        own_bf16[...] = own_src[...].astype(jnp.bfloat16)
        cp_out_own = pltpu.make_async_copy(
            own_bf16, out_hbm.at[pl.ds(my_x * m, m)], lsems.at[2],
        )
        cp_out_own.start()

        y_rdmas = []
        cp_locs = []
        for b in range(NB):
            br = BLOCKS[b]
            x_rdmas[b].wait_recv()
            rdma = pltpu.make_async_remote_copy(
                src_ref=xbuf.at[pl.ds(OFFS[b], br)],
                dst_ref=out_hbm.at[pl.ds(recv_base + OFFS[b], br)],
                send_sem=y_send_sems.at[b],
                recv_sem=y_recv_sems.at[b],
                device_id=(my_x, other_y),
                device_id_type=pl.DeviceIdType.MESH,
            )
            rdma.start()
            y_rdmas.append(rdma)
            cp = pltpu.make_async_copy(
                xbuf.at[pl.ds(OFFS[b], br)],
                out_hbm.at[pl.ds(recv_base + OFFS[b], br)],
                lcp_sems.at[b],
            )
            cp.start()
            cp_locs.append(cp)

        for b in range(NB):
            y_rdmas[b].wait_recv()
        cp_out_own.wait()
        for b in range(NB):
            cp_locs[b].wait()
        for b in range(NB):
            x_rdmas[b].wait_send()
            y_rdmas[b].wait_send()

    out_shape = jax.ShapeDtypeStruct((2 * m, out_cols), jnp.bfloat16)
    return pl.pallas_call(
        body,
        out_shape=out_shape,
        in_specs=[pl.BlockSpec(memory_space=pl.ANY)],
        out_specs=pl.BlockSpec(memory_space=pl.ANY),
        scratch_shapes=[
            pltpu.VMEM((half, out_cols), jnp.float32),
            pltpu.VMEM((m, out_cols), jnp.float32),
            pltpu.VMEM((m, out_cols), jnp.bfloat16),
            pltpu.VMEM((half, out_cols), jnp.bfloat16),
            pltpu.VMEM((half, out_cols), jnp.bfloat16),
            pltpu.SemaphoreType.DMA((3,)),
            pltpu.SemaphoreType.DMA((NB,)),
            pltpu.SemaphoreType.DMA((NB,)),
            pltpu.SemaphoreType.DMA((NB,)),
            pltpu.SemaphoreType.DMA((NB,)),
            pltpu.SemaphoreType.DMA((NB,)),
        ],
        compiler_params=pltpu.CompilerParams(collective_id=0),
    )(x)
